# baseline (speedup 1.0000x reference)
import functools

import jax
import jax.numpy as jnp
from jax import lax
from jax.experimental import pallas as pl
from jax.experimental.pallas import tpu as pltpu
from jax.experimental.pallas import tpu_sc as plsc

C = 16
NC = 2
NS = 16
NW = NC * NS
MOMENTUM = 0.9
ALPHA = 0.5
LN2 = 0.6931471805599453

LANES = 128
SUB = 8
CT = 16
NBUF = 2


def _log_f32(s):
    bits = lax.bitcast_convert_type(s, jnp.int32)
    e = lax.shift_right_logical(bits, 23) - 127
    mant_bits = lax.bitwise_or(lax.bitwise_and(bits, 0x007FFFFF), 0x3F800000)
    mf = lax.bitcast_convert_type(mant_bits, jnp.float32)
    z = (mf - 1.0) / (mf + 1.0)
    z2 = z * z
    poly = 1.0 + z2 * (1.0 / 3.0 + z2 * (1.0 / 5.0 + z2 * (1.0 / 7.0)))
    return e.astype(jnp.float32) * LN2 + 2.0 * z * poly


def _sc_partials(y4, y_true, nt_sc):
    tiles_per_w = nt_sc // NW
    n_chunks = tiles_per_w // CT
    groups_per_chunk = CT * (LANES // C)

    mesh = plsc.VectorSubcoreMesh(core_axis_name="c", subcore_axis_name="s")

    @functools.partial(
        pl.kernel,
        out_type=jax.ShapeDtypeStruct((NW, 4 * C), jnp.float32),
        mesh=mesh,
        compiler_params=pltpu.CompilerParams(
            needs_layout_passes=False, use_tc_tiling_on_sc=False),
        scratch_types=[
            pltpu.VMEM((NBUF, 2, CT, SUB, LANES), jnp.float32),
            pltpu.VMEM((NBUF, CT * LANES,), jnp.int32),
            pltpu.VMEM((2 * 4 * C,), jnp.float32),
            pltpu.SemaphoreType.DMA,
            pltpu.SemaphoreType.DMA,
            pltpu.SemaphoreType.DMA,
            pltpu.SemaphoreType.DMA,
        ],
    )
    def sc_kernel(y_hbm, t_hbm, out_hbm, yv, tv, accv,
                  ysem0, ysem1, tsem0, tsem1):
        cid = lax.axis_index("c")
        sid = lax.axis_index("s")
        wid = sid * NC + cid
        tile0 = wid * tiles_per_w
        ysems = (ysem0, ysem1)
        tsems = (tsem0, tsem1)

        zeros = jnp.zeros((C,), jnp.float32)
        ones = jnp.ones((C,), jnp.float32)
        lane = lax.iota(jnp.int32, C)
        for k in range(8):
            accv[pl.ds(k * C, C)] = zeros

        def ycopy(ci, b):
            t0 = tile0 + ci * CT
            return pltpu.make_async_copy(
                y_hbm.at[:, pl.ds(t0, CT), :, :], yv.at[b], ysems[b])

        def tcopy(ci, b):
            t0 = tile0 + ci * CT
            return pltpu.make_async_copy(
                t_hbm.at[pl.ds(t0 * LANES, CT * LANES)], tv.at[b], tsems[b])

        for b in range(NBUF):
            ycopy(b, b).start()
            tcopy(b, b).start()

        def process_chunk(ci, b):
            ycopy(ci, b).wait()
            tcopy(ci, b).wait()

            def group_body(g, carry):
                tc = lax.shift_right_logical(g, 3)
                l0 = lax.bitwise_and(g, 7) * C
                tcv = jnp.full((C,), 0, jnp.int32) + tc
                if True:
                    t = tv[b, pl.ds(g * C, C)]
                    us = [yv[b, c // SUB, tc, c % SUB, pl.ds(l0, C)]
                          for c in range(C)]
                    m = us
                    while len(m) > 1:
                        m = [jnp.maximum(m[2 * i], m[2 * i + 1])
                             for i in range(len(m) // 2)]
                    m = m[0]
                    idx = [jnp.where(us[c] == m, jnp.int32(c), jnp.int32(C))
                           for c in range(C)]
                    while len(idx) > 1:
                        idx = [jnp.minimum(idx[2 * i], idx[2 * i + 1])
                               for i in range(len(idx) // 2)]
                    pred = idx[0]
                    e = [jnp.exp(us[c] - m) for c in range(C)]
                    while len(e) > 1:
                        e = [e[2 * i] + e[2 * i + 1]
                             for i in range(len(e) // 2)]
                    s = e[0]
                    tval = plsc.load_gather(
                        yv.at[b],
                        [lax.shift_right_logical(t, 3), tcv,
                         lax.bitwise_and(t, 7), l0 + lane])
                    logp = tval - m - _log_f32(s)
                    plsc.addupdate_scatter(accv, [t], ones)
                    plsc.addupdate_scatter(accv, [pred + C], ones)
                    plsc.addupdate_scatter(accv, [pred + 2 * C], ones,
                                           mask=pred == t)
                    plsc.addupdate_scatter(accv, [t + 3 * C], logp)
                return carry

            lax.fori_loop(0, groups_per_chunk, group_body, 0, unroll=2)

        def chunk_pair(ci2, _):
            for b in range(NBUF):
                ci = ci2 * NBUF + b
                process_chunk(ci, b)
                nci = ci + NBUF

                @pl.when(nci < n_chunks)
                def _start_next():
                    ycopy(nci, b).start()
                    tcopy(nci, b).start()
            return _

        lax.fori_loop(0, n_chunks // NBUF, chunk_pair, 0, unroll=False)
        pltpu.sync_copy(accv.at[pl.ds(0, 4 * C)], out_hbm.at[wid])

    return sc_kernel(y4, y_true)


TC_BLK = 512
NT_SC = 3072


TC_GRPS = 32
TC_W = TC_BLK * TC_GRPS
TC_NBUF = 4


def _make_tc_body(n_steps):
    def _tc_body(yt_hbm, t2_hbm, o_ref, xv, tv,
                 xs0, xs1, xs2, xs3, ts0, ts1, ts2, ts3):
        xsems = (xs0, xs1, xs2, xs3)
        tsems = (ts0, ts1, ts2, ts3)
        col0 = NT_SC * LANES

        def xcopy(ci, b):
            return pltpu.make_async_copy(
                yt_hbm.at[:, pl.ds(col0 + ci * TC_W, TC_W)],
                xv.at[b], xsems[b])

        def tcopy(ci, b):
            return pltpu.make_async_copy(
                t2_hbm.at[pl.ds(col0 // TC_BLK + ci * TC_GRPS, TC_GRPS), :],
                tv.at[b], tsems[b])

        for b in range(TC_NBUF):
            xcopy(b, b).start()
            tcopy(b, b).start()

        o_ref[...] = jnp.zeros_like(o_ref)
        cls = lax.broadcasted_iota(jnp.int32, (C, TC_BLK), 0)

        def process(ci, b):
            xcopy(ci, b).wait()
            tcopy(ci, b).wait()
            a_rec = a_prc = a_rgt = a_sum = jnp.zeros(
                (C, TC_BLK), jnp.float32)
            for r in range(TC_GRPS):
                x = xv[b, :, pl.ds(r * TC_BLK, TC_BLK)]
                tr = tv[b, pl.ds(r, 1), :]
                m = jnp.max(x, axis=0, keepdims=True)
                e = jnp.exp(x - m)
                s = jnp.sum(e, axis=0, keepdims=True)
                lse = m + jnp.log(s)
                idx = jnp.where(x == m, cls, C)
                pred = jnp.min(idx, axis=0, keepdims=True)
                oh_t = (cls == tr).astype(jnp.float32)
                oh_p = (cls == pred).astype(jnp.float32)
                rightm = (pred == tr).astype(jnp.float32)
                a_rec = a_rec + oh_t
                a_prc = a_prc + oh_p
                a_rgt = a_rgt + oh_p * rightm
                a_sum = a_sum + oh_t * (x - lse)
            o_ref[pl.ds(0 * C, C), :] += a_rec
            o_ref[pl.ds(1 * C, C), :] += a_prc
            o_ref[pl.ds(2 * C, C), :] += a_rgt
            o_ref[pl.ds(3 * C, C), :] += a_sum

        def step_pair(ci2, carry):
            for b in range(TC_NBUF):
                ci = ci2 * TC_NBUF + b
                process(ci, b)
                nci = ci + TC_NBUF

                @pl.when(nci < n_steps)
                def _start_next():
                    xcopy(nci, b).start()
                    tcopy(nci, b).start()
            return carry

        lax.fori_loop(0, n_steps // TC_NBUF, step_pair, 0, unroll=False)

    return _tc_body


def _tc_partials(yt, y_true2, nt_tc):
    n_steps = nt_tc * LANES // TC_W
    return pl.pallas_call(
        _make_tc_body(n_steps),
        in_specs=[
            pl.BlockSpec(memory_space=pl.ANY),
            pl.BlockSpec(memory_space=pl.ANY),
        ],
        out_specs=pl.BlockSpec((4 * C, TC_BLK), lambda: (0, 0)),
        out_shape=jax.ShapeDtypeStruct((4 * C, TC_BLK), jnp.float32),
        scratch_shapes=[
            pltpu.VMEM((TC_NBUF, C, TC_W), jnp.float32),
            pltpu.VMEM((TC_NBUF, TC_GRPS, TC_BLK), jnp.int32),
        ] + [pltpu.SemaphoreType.DMA] * (2 * TC_NBUF),
    )(yt, y_true2)


def _epilogue_kernel(parts_ref, tc_ref, o_ref):
    parts = parts_ref[...]
    acc = jnp.sum(parts, axis=0, keepdims=True)
    tcp = jnp.sum(tc_ref[...], axis=1)
    acc = acc + tcp[None, :]
    rec = acc[:, 0:C]
    prc = acc[:, C:2 * C]
    rgt = acc[:, 2 * C:3 * C]
    ssum = acc[:, 3 * C:4 * C]
    p = rgt / prc
    r = rgt / rec
    focal = 1.0 - p * r / (ALPHA * p + (1.0 - ALPHA) * r)
    w = (1.0 - MOMENTUM) * focal
    num = jnp.sum(w * ssum)
    den = jnp.sum(w * rec)
    o_ref[0, 0] = -num / den


def kernel(y, y_true):
    batch, c = y.shape
    nt = batch // LANES
    y4 = jnp.swapaxes(y.T.reshape(c // SUB, SUB, batch // LANES, LANES), 1, 2)
    yt = y.T
    y_true2 = y_true.reshape(batch // TC_BLK, TC_BLK)
    parts = _sc_partials(y4, y_true, NT_SC)
    tc_parts = _tc_partials(yt, y_true2, nt - NT_SC)
    loss = pl.pallas_call(
        _epilogue_kernel,
        out_shape=jax.ShapeDtypeStruct((1, 1), jnp.float32),
        out_specs=pl.BlockSpec(memory_space=pltpu.SMEM),
    )(parts, tc_parts)
    return loss[0, 0]


# speedup vs baseline: 1.0009x; 1.0009x over previous
import functools

import jax
import jax.numpy as jnp
from jax import lax
from jax.experimental import pallas as pl
from jax.experimental.pallas import tpu as pltpu
from jax.experimental.pallas import tpu_sc as plsc

C = 16
NC = 2
NS = 16
NW = NC * NS
MOMENTUM = 0.9
ALPHA = 0.5
LN2 = 0.6931471805599453

LANES = 128
SUB = 8
CT = 16
NBUF = 2


def _log_f32(s):
    bits = lax.bitcast_convert_type(s, jnp.int32)
    e = lax.shift_right_logical(bits, 23) - 127
    mant_bits = lax.bitwise_or(lax.bitwise_and(bits, 0x007FFFFF), 0x3F800000)
    mf = lax.bitcast_convert_type(mant_bits, jnp.float32)
    z = (mf - 1.0) / (mf + 1.0)
    z2 = z * z
    poly = 1.0 + z2 * (1.0 / 3.0 + z2 * (1.0 / 5.0 + z2 * (1.0 / 7.0)))
    return e.astype(jnp.float32) * LN2 + 2.0 * z * poly


def _sc_partials(y4, y_true, nt_sc):
    tiles_per_w = nt_sc // NW
    n_chunks = tiles_per_w // CT
    groups_per_chunk = CT * (LANES // C)

    mesh = plsc.VectorSubcoreMesh(core_axis_name="c", subcore_axis_name="s")

    @functools.partial(
        pl.kernel,
        out_type=jax.ShapeDtypeStruct((NW, 4 * C), jnp.float32),
        mesh=mesh,
        compiler_params=pltpu.CompilerParams(
            needs_layout_passes=False, use_tc_tiling_on_sc=False),
        scratch_types=[
            pltpu.VMEM((NBUF, 2, CT, SUB, LANES), jnp.float32),
            pltpu.VMEM((NBUF, CT * LANES,), jnp.int32),
            pltpu.VMEM((2 * 4 * C,), jnp.float32),
            pltpu.SemaphoreType.DMA,
            pltpu.SemaphoreType.DMA,
            pltpu.SemaphoreType.DMA,
            pltpu.SemaphoreType.DMA,
        ],
    )
    def sc_kernel(y_hbm, t_hbm, out_hbm, yv, tv, accv,
                  ysem0, ysem1, tsem0, tsem1):
        cid = lax.axis_index("c")
        sid = lax.axis_index("s")
        wid = sid * NC + cid
        tile0 = wid * tiles_per_w
        ysems = (ysem0, ysem1)
        tsems = (tsem0, tsem1)

        zeros = jnp.zeros((C,), jnp.float32)
        ones = jnp.ones((C,), jnp.float32)
        lane = lax.iota(jnp.int32, C)
        for k in range(8):
            accv[pl.ds(k * C, C)] = zeros

        def ycopy(ci, b):
            t0 = tile0 + ci * CT
            return pltpu.make_async_copy(
                y_hbm.at[:, pl.ds(t0, CT), :, :], yv.at[b], ysems[b])

        def tcopy(ci, b):
            t0 = tile0 + ci * CT
            return pltpu.make_async_copy(
                t_hbm.at[pl.ds(t0 * LANES, CT * LANES)], tv.at[b], tsems[b])

        for b in range(NBUF):
            ycopy(b, b).start()
            tcopy(b, b).start()

        def process_chunk(ci, b):
            ycopy(ci, b).wait()
            tcopy(ci, b).wait()

            def group_body(g, carry):
                tc = lax.shift_right_logical(g, 3)
                l0 = lax.bitwise_and(g, 7) * C
                tcv = jnp.full((C,), 0, jnp.int32) + tc
                if True:
                    t = tv[b, pl.ds(g * C, C)]
                    us = [yv[b, c // SUB, tc, c % SUB, pl.ds(l0, C)]
                          for c in range(C)]
                    m = us
                    while len(m) > 1:
                        m = [jnp.maximum(m[2 * i], m[2 * i + 1])
                             for i in range(len(m) // 2)]
                    m = m[0]
                    idx = [jnp.where(us[c] == m, jnp.int32(c), jnp.int32(C))
                           for c in range(C)]
                    while len(idx) > 1:
                        idx = [jnp.minimum(idx[2 * i], idx[2 * i + 1])
                               for i in range(len(idx) // 2)]
                    pred = idx[0]
                    e = [jnp.exp(us[c] - m) for c in range(C)]
                    while len(e) > 1:
                        e = [e[2 * i] + e[2 * i + 1]
                             for i in range(len(e) // 2)]
                    s = e[0]
                    tval = plsc.load_gather(
                        yv.at[b],
                        [lax.shift_right_logical(t, 3), tcv,
                         lax.bitwise_and(t, 7), l0 + lane])
                    logp = tval - m - _log_f32(s)
                    plsc.addupdate_scatter(accv, [t], ones)
                    plsc.addupdate_scatter(accv, [pred + C], ones)
                    plsc.addupdate_scatter(accv, [pred + 2 * C], ones,
                                           mask=pred == t)
                    plsc.addupdate_scatter(accv, [t + 3 * C], logp)
                return carry

            lax.fori_loop(0, groups_per_chunk, group_body, 0, unroll=3)

        def chunk_pair(ci2, _):
            for b in range(NBUF):
                ci = ci2 * NBUF + b
                process_chunk(ci, b)
                nci = ci + NBUF

                @pl.when(nci < n_chunks)
                def _start_next():
                    ycopy(nci, b).start()
                    tcopy(nci, b).start()
            return _

        lax.fori_loop(0, n_chunks // NBUF, chunk_pair, 0, unroll=False)
        pltpu.sync_copy(accv.at[pl.ds(0, 4 * C)], out_hbm.at[wid])

    return sc_kernel(y4, y_true)


TC_BLK = 512
NT_SC = 3072


TC_GRPS = 32
TC_W = TC_BLK * TC_GRPS
TC_NBUF = 2


def _make_tc_body(n_steps):
    def _tc_body(yt_hbm, t2_hbm, o_ref, xv, tv, xs0, xs1, ts0, ts1):
        xsems = (xs0, xs1)
        tsems = (ts0, ts1)
        col0 = NT_SC * LANES

        def xcopy(ci, b):
            return pltpu.make_async_copy(
                yt_hbm.at[:, pl.ds(col0 + ci * TC_W, TC_W)],
                xv.at[b], xsems[b])

        def tcopy(ci, b):
            return pltpu.make_async_copy(
                t2_hbm.at[pl.ds(col0 // TC_BLK + ci * TC_GRPS, TC_GRPS), :],
                tv.at[b], tsems[b])

        for b in range(TC_NBUF):
            xcopy(b, b).start()
            tcopy(b, b).start()

        o_ref[...] = jnp.zeros_like(o_ref)
        cls = lax.broadcasted_iota(jnp.int32, (C, TC_BLK), 0)

        def process(ci, b):
            xcopy(ci, b).wait()
            tcopy(ci, b).wait()
            a_rec = a_prc = a_rgt = a_sum = jnp.zeros(
                (C, TC_BLK), jnp.float32)
            for r in range(TC_GRPS):
                x = xv[b, :, pl.ds(r * TC_BLK, TC_BLK)]
                tr = tv[b, pl.ds(r, 1), :]
                m = jnp.max(x, axis=0, keepdims=True)
                e = jnp.exp(x - m)
                s = jnp.sum(e, axis=0, keepdims=True)
                lse = m + jnp.log(s)
                idx = jnp.where(x == m, cls, C)
                pred = jnp.min(idx, axis=0, keepdims=True)
                oh_t = (cls == tr).astype(jnp.float32)
                oh_p = (cls == pred).astype(jnp.float32)
                rightm = (pred == tr).astype(jnp.float32)
                a_rec = a_rec + oh_t
                a_prc = a_prc + oh_p
                a_rgt = a_rgt + oh_p * rightm
                a_sum = a_sum + oh_t * (x - lse)
            o_ref[pl.ds(0 * C, C), :] += a_rec
            o_ref[pl.ds(1 * C, C), :] += a_prc
            o_ref[pl.ds(2 * C, C), :] += a_rgt
            o_ref[pl.ds(3 * C, C), :] += a_sum

        def step_pair(ci2, carry):
            for b in range(TC_NBUF):
                ci = ci2 * TC_NBUF + b
                process(ci, b)
                nci = ci + TC_NBUF

                @pl.when(nci < n_steps)
                def _start_next():
                    xcopy(nci, b).start()
                    tcopy(nci, b).start()
            return carry

        lax.fori_loop(0, n_steps // TC_NBUF, step_pair, 0, unroll=False)

    return _tc_body


def _tc_partials(yt, y_true2, nt_tc):
    n_steps = nt_tc * LANES // TC_W
    return pl.pallas_call(
        _make_tc_body(n_steps),
        in_specs=[
            pl.BlockSpec(memory_space=pl.ANY),
            pl.BlockSpec(memory_space=pl.ANY),
        ],
        out_specs=pl.BlockSpec((4 * C, TC_BLK), lambda: (0, 0)),
        out_shape=jax.ShapeDtypeStruct((4 * C, TC_BLK), jnp.float32),
        scratch_shapes=[
            pltpu.VMEM((TC_NBUF, C, TC_W), jnp.float32),
            pltpu.VMEM((TC_NBUF, TC_GRPS, TC_BLK), jnp.int32),
        ] + [pltpu.SemaphoreType.DMA] * (2 * TC_NBUF),
    )(yt, y_true2)


def _epilogue_kernel(parts_ref, tc_ref, o_ref):
    parts = parts_ref[...]
    acc = jnp.sum(parts, axis=0, keepdims=True)
    tcp = jnp.sum(tc_ref[...], axis=1)
    acc = acc + tcp[None, :]
    rec = acc[:, 0:C]
    prc = acc[:, C:2 * C]
    rgt = acc[:, 2 * C:3 * C]
    ssum = acc[:, 3 * C:4 * C]
    p = rgt / prc
    r = rgt / rec
    focal = 1.0 - p * r / (ALPHA * p + (1.0 - ALPHA) * r)
    w = (1.0 - MOMENTUM) * focal
    num = jnp.sum(w * ssum)
    den = jnp.sum(w * rec)
    o_ref[0, 0] = -num / den


def kernel(y, y_true):
    batch, c = y.shape
    nt = batch // LANES
    y4 = jnp.swapaxes(y.T.reshape(c // SUB, SUB, batch // LANES, LANES), 1, 2)
    yt = y.T
    y_true2 = y_true.reshape(batch // TC_BLK, TC_BLK)
    parts = _sc_partials(y4, y_true, NT_SC)
    tc_parts = _tc_partials(yt, y_true2, nt - NT_SC)
    loss = pl.pallas_call(
        _epilogue_kernel,
        out_shape=jax.ShapeDtypeStruct((1, 1), jnp.float32),
        out_specs=pl.BlockSpec(memory_space=pltpu.SMEM),
    )(parts, tc_parts)
    return loss[0, 0]


# speedup vs baseline: 1.0124x; 1.0114x over previous
import functools

import jax
import jax.numpy as jnp
from jax import lax
from jax.experimental import pallas as pl
from jax.experimental.pallas import tpu as pltpu
from jax.experimental.pallas import tpu_sc as plsc

C = 16
NC = 2
NS = 16
NW = NC * NS
MOMENTUM = 0.9
ALPHA = 0.5
LN2 = 0.6931471805599453

LANES = 128
SUB = 8
CT = 16
NBUF = 2


def _log_f32(s):
    bits = lax.bitcast_convert_type(s, jnp.int32)
    e = lax.shift_right_logical(bits, 23) - 127
    mant_bits = lax.bitwise_or(lax.bitwise_and(bits, 0x007FFFFF), 0x3F800000)
    mf = lax.bitcast_convert_type(mant_bits, jnp.float32)
    z = (mf - 1.0) / (mf + 1.0)
    z2 = z * z
    poly = 1.0 + z2 * (1.0 / 3.0 + z2 * (1.0 / 5.0 + z2 * (1.0 / 7.0)))
    return e.astype(jnp.float32) * LN2 + 2.0 * z * poly


def _sc_partials(y4, y_true, nt_sc):
    tiles_per_w = nt_sc // NW
    n_chunks = tiles_per_w // CT
    groups_per_chunk = CT * (LANES // C)

    mesh = plsc.VectorSubcoreMesh(core_axis_name="c", subcore_axis_name="s")

    @functools.partial(
        pl.kernel,
        out_type=jax.ShapeDtypeStruct((NW, 4 * C), jnp.float32),
        mesh=mesh,
        compiler_params=pltpu.CompilerParams(
            needs_layout_passes=False, use_tc_tiling_on_sc=False),
        scratch_types=[
            pltpu.VMEM((NBUF, 2, CT, SUB, LANES), jnp.float32),
            pltpu.VMEM((NBUF, CT * LANES,), jnp.int32),
            pltpu.VMEM((2 * 4 * C,), jnp.float32),
            pltpu.SemaphoreType.DMA,
            pltpu.SemaphoreType.DMA,
            pltpu.SemaphoreType.DMA,
            pltpu.SemaphoreType.DMA,
        ],
    )
    def sc_kernel(y_hbm, t_hbm, out_hbm, yv, tv, accv,
                  ysem0, ysem1, tsem0, tsem1):
        cid = lax.axis_index("c")
        sid = lax.axis_index("s")
        wid = sid * NC + cid
        tile0 = wid * tiles_per_w
        ysems = (ysem0, ysem1)
        tsems = (tsem0, tsem1)

        zeros = jnp.zeros((C,), jnp.float32)
        ones = jnp.ones((C,), jnp.float32)
        lane = lax.iota(jnp.int32, C)
        for k in range(8):
            accv[pl.ds(k * C, C)] = zeros

        def ycopy(ci, b):
            t0 = tile0 + ci * CT
            return pltpu.make_async_copy(
                y_hbm.at[:, pl.ds(t0, CT), :, :], yv.at[b], ysems[b])

        def tcopy(ci, b):
            t0 = tile0 + ci * CT
            return pltpu.make_async_copy(
                t_hbm.at[pl.ds(t0 * LANES, CT * LANES)], tv.at[b], tsems[b])

        for b in range(NBUF):
            ycopy(b, b).start()
            tcopy(b, b).start()

        def process_chunk(ci, b):
            ycopy(ci, b).wait()
            tcopy(ci, b).wait()

            def group_body(g, carry):
                tc = lax.shift_right_logical(g, 3)
                l0 = lax.bitwise_and(g, 7) * C
                tcv = jnp.full((C,), 0, jnp.int32) + tc
                if True:
                    t = tv[b, pl.ds(g * C, C)]
                    us = [yv[b, c // SUB, tc, c % SUB, pl.ds(l0, C)]
                          for c in range(C)]
                    m = us
                    while len(m) > 1:
                        m = [jnp.maximum(m[2 * i], m[2 * i + 1])
                             for i in range(len(m) // 2)]
                    m = m[0]
                    idx = [jnp.where(us[c] == m, jnp.int32(c), jnp.int32(C))
                           for c in range(C)]
                    while len(idx) > 1:
                        idx = [jnp.minimum(idx[2 * i], idx[2 * i + 1])
                               for i in range(len(idx) // 2)]
                    pred = idx[0]
                    e = [jnp.exp(us[c] - m) for c in range(C)]
                    while len(e) > 1:
                        e = [e[2 * i] + e[2 * i + 1]
                             for i in range(len(e) // 2)]
                    s = e[0]
                    tval = plsc.load_gather(
                        yv.at[b],
                        [lax.shift_right_logical(t, 3), tcv,
                         lax.bitwise_and(t, 7), l0 + lane])
                    logp = tval - m - _log_f32(s)
                    plsc.addupdate_scatter(accv, [t], ones)
                    plsc.addupdate_scatter(accv, [pred + C], ones)
                    plsc.addupdate_scatter(accv, [pred + 2 * C], ones,
                                           mask=pred == t)
                    plsc.addupdate_scatter(accv, [t + 3 * C], logp)
                return carry

            lax.fori_loop(0, groups_per_chunk, group_body, 0, unroll=2)

        def chunk_pair(ci2, _):
            for b in range(NBUF):
                ci = ci2 * NBUF + b
                process_chunk(ci, b)
                nci = ci + NBUF

                @pl.when(nci < n_chunks)
                def _start_next():
                    ycopy(nci, b).start()
                    tcopy(nci, b).start()
            return _

        lax.fori_loop(0, n_chunks // NBUF, chunk_pair, 0, unroll=False)
        pltpu.sync_copy(accv.at[pl.ds(0, 4 * C)], out_hbm.at[wid])

    return sc_kernel(y4, y_true)


TC_BLK = 512
NT_SC = 3072


TC_GRPS = 32
TC_W = TC_BLK * TC_GRPS
TC_NBUF = 2


def _make_tc_body(n_steps):
    def _tc_body(yt_hbm, t2_hbm, o_ref, xv, tv, accp, xs0, xs1, ts0, ts1):
        xsems = (xs0, xs1)
        tsems = (ts0, ts1)
        col0 = NT_SC * LANES

        def xcopy(ci, b):
            return pltpu.make_async_copy(
                yt_hbm.at[:, pl.ds(col0 + ci * TC_W, TC_W)],
                xv.at[b], xsems[b])

        def tcopy(ci, b):
            return pltpu.make_async_copy(
                t2_hbm.at[pl.ds(col0 // TC_BLK + ci * TC_GRPS, TC_GRPS), :],
                tv.at[b], tsems[b])

        for b in range(TC_NBUF):
            xcopy(b, b).start()
            tcopy(b, b).start()

        accp[...] = jnp.zeros_like(accp)
        cls = lax.broadcasted_iota(jnp.int32, (C, TC_BLK), 0)

        def process(ci, b):
            xcopy(ci, b).wait()
            tcopy(ci, b).wait()
            a_rec = a_prc = a_rgt = a_sum = jnp.zeros(
                (C, TC_BLK), jnp.float32)
            for r in range(TC_GRPS):
                x = xv[b, :, pl.ds(r * TC_BLK, TC_BLK)]
                tr = tv[b, pl.ds(r, 1), :]
                m = jnp.max(x, axis=0, keepdims=True)
                e = jnp.exp(x - m)
                s = jnp.sum(e, axis=0, keepdims=True)
                lse = m + jnp.log(s)
                idx = jnp.where(x == m, cls, C)
                pred = jnp.min(idx, axis=0, keepdims=True)
                oh_t = (cls == tr).astype(jnp.float32)
                oh_p = (cls == pred).astype(jnp.float32)
                rightm = (pred == tr).astype(jnp.float32)
                a_rec = a_rec + oh_t
                a_prc = a_prc + oh_p
                a_rgt = a_rgt + oh_p * rightm
                a_sum = a_sum + oh_t * (x - lse)
            accp[pl.ds(0 * C, C), :] += a_rec
            accp[pl.ds(1 * C, C), :] += a_prc
            accp[pl.ds(2 * C, C), :] += a_rgt
            accp[pl.ds(3 * C, C), :] += a_sum

        def step_pair(ci2, carry):
            for b in range(TC_NBUF):
                ci = ci2 * TC_NBUF + b
                process(ci, b)
                nci = ci + TC_NBUF

                @pl.when(nci < n_steps)
                def _start_next():
                    xcopy(nci, b).start()
                    tcopy(nci, b).start()
            return carry

        lax.fori_loop(0, n_steps // TC_NBUF, step_pair, 0, unroll=False)
        for k in range(4):
            o_ref[0, pl.ds(k * C, C)] = jnp.sum(
                accp[pl.ds(k * C, C), :], axis=1)

    return _tc_body


def _tc_partials(yt, y_true2, nt_tc):
    n_steps = nt_tc * LANES // TC_W
    return pl.pallas_call(
        _make_tc_body(n_steps),
        in_specs=[
            pl.BlockSpec(memory_space=pl.ANY),
            pl.BlockSpec(memory_space=pl.ANY),
        ],
        out_specs=pl.BlockSpec((1, 4 * C), lambda: (0, 0)),
        out_shape=jax.ShapeDtypeStruct((1, 4 * C), jnp.float32),
        scratch_shapes=[
            pltpu.VMEM((TC_NBUF, C, TC_W), jnp.float32),
            pltpu.VMEM((TC_NBUF, TC_GRPS, TC_BLK), jnp.int32),
            pltpu.VMEM((4 * C, TC_BLK), jnp.float32),
        ] + [pltpu.SemaphoreType.DMA] * (2 * TC_NBUF),
    )(yt, y_true2)


def _epilogue_kernel(parts_ref, tc_ref, o_ref):
    parts = parts_ref[...]
    acc = jnp.sum(parts, axis=0, keepdims=True)
    acc = acc + tc_ref[...]
    rec = acc[:, 0:C]
    prc = acc[:, C:2 * C]
    rgt = acc[:, 2 * C:3 * C]
    ssum = acc[:, 3 * C:4 * C]
    p = rgt / prc
    r = rgt / rec
    focal = 1.0 - p * r / (ALPHA * p + (1.0 - ALPHA) * r)
    w = (1.0 - MOMENTUM) * focal
    num = jnp.sum(w * ssum)
    den = jnp.sum(w * rec)
    o_ref[0, 0] = -num / den


def kernel(y, y_true):
    batch, c = y.shape
    nt = batch // LANES
    y4 = jnp.swapaxes(y.T.reshape(c // SUB, SUB, batch // LANES, LANES), 1, 2)
    yt = y.T
    y_true2 = y_true.reshape(batch // TC_BLK, TC_BLK)
    parts = _sc_partials(y4, y_true, NT_SC)
    tc_parts = _tc_partials(yt, y_true2, nt - NT_SC)
    loss = pl.pallas_call(
        _epilogue_kernel,
        out_shape=jax.ShapeDtypeStruct((1, 1), jnp.float32),
        out_specs=pl.BlockSpec(memory_space=pltpu.SMEM),
    )(parts, tc_parts)
    return loss[0, 0]
